# baseline (speedup 1.0000x reference)
import jax
import jax.numpy as jnp
from jax.experimental import pallas as pl
from jax.experimental.pallas import tpu as pltpu

_BATCH = 16384
_INPUT_DIM = 1000
_EMBED_DIM = 16
_BM = 512


def _mm_body(x_ref, e_ref, o_ref):
    o_ref[...] = jnp.dot(x_ref[...], e_ref[...],
                         preferred_element_type=jnp.float32)


def kernel(x, embedding):
    grid = (_BATCH // _BM,)
    return pl.pallas_call(
        _mm_body,
        grid=grid,
        in_specs=[
            pl.BlockSpec((_BM, _INPUT_DIM), lambda i: (i, 0)),
            pl.BlockSpec((_INPUT_DIM, _EMBED_DIM), lambda i: (0, 0)),
        ],
        out_specs=pl.BlockSpec((_BM, _EMBED_DIM), lambda i: (i, 0)),
        out_shape=jax.ShapeDtypeStruct((_BATCH, _EMBED_DIM), jnp.float32),
        compiler_params=pltpu.CompilerParams(
            dimension_semantics=("parallel",),
        ),
    )(x, embedding)


# speedup vs baseline: 1.1401x; 1.1401x over previous
import jax
import jax.numpy as jnp
from jax.experimental import pallas as pl
from jax.experimental.pallas import tpu as pltpu

_BATCH = 16384
_INPUT_DIM = 1000
_EMBED_DIM = 16
_NSPLIT = 4
_BM = 512
_QROWS = _BATCH // _NSPLIT


def _mm_body(x0, x1, x2, x3, e_ref, o_ref):
    for r, xr in enumerate((x0, x1, x2, x3)):
        o_ref[r] = jnp.dot(xr[...], e_ref[...],
                           preferred_element_type=jnp.float32)


def kernel(x, embedding):
    grid = (_QROWS // _BM,)

    def xspec(r):
        return pl.BlockSpec((_BM, _INPUT_DIM),
                            lambda i, r=r: (r * (_QROWS // _BM) + i, 0))

    out = pl.pallas_call(
        _mm_body,
        grid=grid,
        in_specs=[xspec(r) for r in range(_NSPLIT)] + [
            pl.BlockSpec((_INPUT_DIM, _EMBED_DIM), lambda i: (0, 0)),
        ],
        out_specs=pl.BlockSpec((_NSPLIT, _BM, _EMBED_DIM),
                               lambda i: (0, i, 0)),
        out_shape=jax.ShapeDtypeStruct((_NSPLIT, _QROWS, _EMBED_DIM),
                                       jnp.float32),
        compiler_params=pltpu.CompilerParams(
            dimension_semantics=("arbitrary",),
        ),
    )(x, x, x, x, embedding)
    return out.reshape(_BATCH, _EMBED_DIM)
